# baseline (speedup 1.0000x reference)
import functools

import jax
import jax.numpy as jnp
from jax import lax
from jax.experimental import pallas as pl
from jax.experimental.pallas import tpu as pltpu
from jax.experimental.pallas import tpu_sc as plsc

NC = 2
NS = 16
NW = NC * NS

N_ATOMS = 100000
D = 512
N0 = 4096
A = 24
TOTAL = N0 * A
PER_TILE = TOTAL // NW
CHUNK = 128
NCHUNK = PER_TILE // CHUNK
ZCH = 6256
TBL = NS * ZCH

SCALE = 1.0 / float(TOTAL)

TROWS = 59040
SROWS = N_ATOMS - TROWS
RPT = SROWS // NW
RC = 80
NCH = RPT // RC
GR = RC // 16
SLAB0 = 58880
CPAD = TROWS - SLAB0
SLABSZ = TBL - SLAB0
CREAD = RPT + 128

_sc_mesh = plsc.VectorSubcoreMesh(core_axis_name="c", subcore_axis_name="s")


@functools.partial(
    pl.kernel,
    out_type=[
        jax.ShapeDtypeStruct((NC, TBL), jnp.float32),
        jax.ShapeDtypeStruct((SLABSZ,), jnp.float32),
        jax.ShapeDtypeStruct((SLABSZ,), jnp.float32),
    ],
    mesh=_sc_mesh,
    scratch_types=[
        pltpu.VMEM((NCHUNK, CHUNK), jnp.int32),
        pltpu.VMEM((CHUNK,), jnp.float32),
        pltpu.VMEM((ZCH,), jnp.float32),
        pltpu.VMEM_SHARED((TBL,), jnp.float32),
        pltpu.SemaphoreType.DMA,
        pltpu.SemaphoreType.DMA,
    ],
)
def _histogram(idx_hbm, out_hbm, outa_hbm, outb_hbm, idx_v, ones_v, zeros_v,
               table_sh, sem_idx, sem_sc):
    cid = lax.axis_index("c")
    sid = lax.axis_index("s")
    w = cid * NS + sid

    idx_cp = pltpu.async_copy(idx_hbm.at[w], idx_v, sem_idx)

    ones16 = jnp.full((16,), 1.0, jnp.float32)
    for i in range(CHUNK // 16):
        ones_v[pl.ds(i * 16, 16)] = ones16

    zero16 = jnp.zeros((16,), jnp.float32)

    def _zbody(i, carry):
        for u in range(17):
            zeros_v[pl.ds((i * 17 + u) * 16, 16)] = zero16
        return carry

    lax.fori_loop(0, ZCH // (16 * 17), _zbody, 0)

    pltpu.sync_copy(zeros_v, table_sh.at[pl.ds(sid * ZCH, ZCH)])
    idx_cp.wait()
    plsc.subcore_barrier()

    cps = [
        pltpu.async_copy(ones_v, table_sh.at[idx_v.at[j]], sem_sc, add=True)
        for j in range(NCHUNK)
    ]
    for cp in cps:
        cp.wait()
    plsc.subcore_barrier()

    @pl.when(sid == 0)
    def _():
        pltpu.sync_copy(table_sh, out_hbm.at[cid])

    @pl.when((sid == 1) & (cid == 0))
    def _():
        pltpu.sync_copy(table_sh.at[pl.ds(SLAB0, SLABSZ)], outa_hbm)

    @pl.when((sid == 1) & (cid == 1))
    def _():
        pltpu.sync_copy(table_sh.at[pl.ds(SLAB0, SLABSZ)], outb_hbm)


BK = 4920
NKB = TROWS // BK


def _splat(vec16, r):
    idx = jnp.full((16, 1), r, dtype=jnp.int32)
    return lax.gather(
        vec16, idx,
        lax.GatherDimensionNumbers(offset_dims=(), collapsed_slice_dims=(0,),
                                   start_index_map=(0,)),
        (1,), mode=lax.GatherScatterMode.PROMISE_IN_BOUNDS)


@functools.partial(
    pl.kernel,
    out_type=jax.ShapeDtypeStruct((NW, D), jnp.float32),
    mesh=_sc_mesh,
    scratch_types=[
        pltpu.VMEM((RC, D), jnp.float32),
        pltpu.VMEM((RC, D), jnp.float32),
        pltpu.VMEM((CREAD,), jnp.float32),
        pltpu.VMEM((CREAD,), jnp.float32),
        pltpu.VMEM((D,), jnp.float32),
        pltpu.SemaphoreType.DMA,
        pltpu.SemaphoreType.DMA,
        pltpu.SemaphoreType.DMA,
    ],
)
def _scmv(cnta_hbm, cntb_hbm, x_hbm, out_hbm, xb0, xb1, c0v, c1v, accv,
          semc, semx0, semx1):
    cid = lax.axis_index("c")
    sid = lax.axis_index("s")
    w = cid * NS + sid
    coff0 = 128 + w * RPT
    base = TROWS + w * RPT

    cc0 = pltpu.async_copy(cnta_hbm.at[pl.ds(coff0, CREAD)], c0v, semc)
    cc1 = pltpu.async_copy(cntb_hbm.at[pl.ds(coff0, CREAD)], c1v, semc)
    pltpu.async_copy(x_hbm.at[pl.ds(base, RC)], xb0, semx0)
    pltpu.async_copy(x_hbm.at[pl.ds(base + RC, RC)], xb1, semx1)
    cc0.wait()
    cc1.wait()

    LSH = CPAD - 128

    def _mbody(i, carry):
        c0v[pl.ds(LSH + i * 16, 16)] = (c0v[pl.ds(LSH + i * 16, 16)]
                                        + c1v[pl.ds(LSH + i * 16, 16)])
        return carry

    lax.fori_loop(0, RPT // 16, _mbody, 0)

    def _chunk(buf, a, accs):
        coff = LSH + a * RC

        def _gbody(g, accs):
            c16 = c0v[pl.ds(coff + g * 16, 16)]
            accs = list(accs)
            for r in range(16):
                s = _splat(c16, r)
                row = g * 16 + r
                for d in range(D // 16):
                    accs[d] = accs[d] + s * buf[row, pl.ds(d * 16, 16)]
            return tuple(accs)

        return plsc.parallel_loop(0, GR, carry=accs, unroll=1)(_gbody)

    def _pbody(t, accs):
        a0 = 2 * t
        pltpu.make_async_copy(x_hbm.at[pl.ds(0, RC)], xb0, semx0).wait()
        accs = _chunk(xb0, a0, accs)

        @pl.when(a0 + 2 < NCH)
        def _():
            pltpu.async_copy(
                x_hbm.at[pl.ds(base + (a0 + 2) * RC, RC)], xb0, semx0)

        pltpu.make_async_copy(x_hbm.at[pl.ds(0, RC)], xb1, semx1).wait()
        accs = _chunk(xb1, a0 + 1, accs)

        @pl.when(a0 + 3 < NCH)
        def _():
            pltpu.async_copy(
                x_hbm.at[pl.ds(base + (a0 + 3) * RC, RC)], xb1, semx1)

        return accs

    accs0 = tuple(jnp.zeros((16,), jnp.float32) for _ in range(D // 16))
    accs = lax.fori_loop(0, NCH // 2, _pbody, accs0)

    for d in range(D // 16):
        accv[pl.ds(d * 16, 16)] = accs[d]
    pltpu.sync_copy(accv, out_hbm.at[w])


def _mv_body(c_ref, x_ref, o_ref):
    k = pl.program_id(0)

    @pl.when(k == 0)
    def _():
        o_ref[...] = jnp.zeros_like(o_ref)

    c = c_ref[0, pl.ds(k, 1), :] + c_ref[1, pl.ds(k, 1), :]
    o_ref[...] += jnp.dot(c, x_ref[...], preferred_element_type=jnp.float32)

    @pl.when(k == NKB - 1)
    def _():
        o_ref[...] = o_ref[...] * SCALE


_matvec = pl.pallas_call(
    _mv_body,
    grid=(NKB,),
    in_specs=[
        pl.BlockSpec((NC, NKB, BK), lambda k: (0, 0, 0)),
        pl.BlockSpec((BK, D), lambda k: (k, 0)),
    ],
    out_specs=pl.BlockSpec((1, D), lambda k: (0, 0)),
    out_shape=jax.ShapeDtypeStruct((1, D), jnp.float32),
)


def kernel(x_atom_fea, node_atom_idx):
    idx = node_atom_idx.astype(jnp.int32).reshape(NW, NCHUNK, CHUNK)
    counts, cnta, cntb = _histogram(idx)
    counts3 = counts[:, :TROWS].reshape(NC, NKB, BK)
    tc = _matvec(counts3, x_atom_fea)
    sc = _scmv(cnta, cntb, x_atom_fea)
    return tc + jnp.sum(sc, axis=0, keepdims=True) * SCALE


# speedup vs baseline: 1.7035x; 1.7035x over previous
import functools

import jax
import jax.numpy as jnp
from jax import lax
from jax.experimental import pallas as pl
from jax.experimental.pallas import tpu as pltpu
from jax.experimental.pallas import tpu_sc as plsc

NC = 2
NS = 16
NW = NC * NS

N_ATOMS = 100000
D = 512
N0 = 4096
A = 24
TOTAL = N0 * A
PER_TILE = TOTAL // NW
CHUNK = 128
NCHUNK = PER_TILE // CHUNK
ZCH = 6256
TBL = NS * ZCH

SCALE = 1.0 / float(TOTAL)

TROWS = 59040
SROWS = N_ATOMS - TROWS
RPT = SROWS // NW
RC = 80
NCH = RPT // RC
GR = RC // 16
SLAB0 = 58880
CPAD = TROWS - SLAB0
SLABSZ = TBL - SLAB0
CREAD = RPT + 128

_sc_mesh = plsc.VectorSubcoreMesh(core_axis_name="c", subcore_axis_name="s")


@functools.partial(
    pl.kernel,
    out_type=[
        jax.ShapeDtypeStruct((NC, TBL), jnp.float32),
        jax.ShapeDtypeStruct((SLABSZ,), jnp.float32),
        jax.ShapeDtypeStruct((SLABSZ,), jnp.float32),
    ],
    mesh=_sc_mesh,
    scratch_types=[
        pltpu.VMEM((NCHUNK, CHUNK), jnp.int32),
        pltpu.VMEM((CHUNK,), jnp.float32),
        pltpu.VMEM((ZCH,), jnp.float32),
        pltpu.VMEM_SHARED((TBL,), jnp.float32),
        pltpu.SemaphoreType.DMA,
        pltpu.SemaphoreType.DMA,
    ],
)
def _histogram(idx_hbm, out_hbm, outa_hbm, outb_hbm, idx_v, ones_v, zeros_v,
               table_sh, sem_idx, sem_sc):
    cid = lax.axis_index("c")
    sid = lax.axis_index("s")
    w = cid * NS + sid

    idx_cp = pltpu.async_copy(idx_hbm.at[w], idx_v, sem_idx)

    ones16 = jnp.full((16,), 1.0, jnp.float32)
    for i in range(CHUNK // 16):
        ones_v[pl.ds(i * 16, 16)] = ones16

    zero16 = jnp.zeros((16,), jnp.float32)

    def _zbody(i, carry):
        for u in range(17):
            zeros_v[pl.ds((i * 17 + u) * 16, 16)] = zero16
        return carry

    lax.fori_loop(0, ZCH // (16 * 17), _zbody, 0)

    pltpu.sync_copy(zeros_v, table_sh.at[pl.ds(sid * ZCH, ZCH)])
    idx_cp.wait()
    plsc.subcore_barrier()

    cps = [
        pltpu.async_copy(ones_v, table_sh.at[idx_v.at[j]], sem_sc, add=True)
        for j in range(NCHUNK)
    ]
    for cp in cps:
        cp.wait()
    plsc.subcore_barrier()

    @pl.when(sid == 0)
    def _():
        pltpu.sync_copy(table_sh, out_hbm.at[cid])

    @pl.when((sid == 1) & (cid == 0))
    def _():
        pltpu.sync_copy(table_sh.at[pl.ds(SLAB0, SLABSZ)], outa_hbm)

    @pl.when((sid == 1) & (cid == 1))
    def _():
        pltpu.sync_copy(table_sh.at[pl.ds(SLAB0, SLABSZ)], outb_hbm)


BK = 4920
NKB = TROWS // BK


def _splat(vec16, r):
    idx = jnp.full((16, 1), r, dtype=jnp.int32)
    return lax.gather(
        vec16, idx,
        lax.GatherDimensionNumbers(offset_dims=(), collapsed_slice_dims=(0,),
                                   start_index_map=(0,)),
        (1,), mode=lax.GatherScatterMode.PROMISE_IN_BOUNDS)


@functools.partial(
    pl.kernel,
    out_type=jax.ShapeDtypeStruct((NW, D), jnp.float32),
    mesh=_sc_mesh,
    scratch_types=[
        pltpu.VMEM((RC, D), jnp.float32),
        pltpu.VMEM((RC, D), jnp.float32),
        pltpu.VMEM((CREAD,), jnp.float32),
        pltpu.VMEM((CREAD,), jnp.float32),
        pltpu.VMEM((D,), jnp.float32),
        pltpu.SemaphoreType.DMA,
        pltpu.SemaphoreType.DMA,
        pltpu.SemaphoreType.DMA,
    ],
)
def _scmv(cnta_hbm, cntb_hbm, x_hbm, out_hbm, xb0, xb1, c0v, c1v, accv,
          semc, semx0, semx1):
    cid = lax.axis_index("c")
    sid = lax.axis_index("s")
    w = cid * NS + sid
    coff0 = 128 + w * RPT
    base = TROWS + w * RPT

    cc0 = pltpu.async_copy(cnta_hbm.at[pl.ds(coff0, CREAD)], c0v, semc)
    cc1 = pltpu.async_copy(cntb_hbm.at[pl.ds(coff0, CREAD)], c1v, semc)
    pltpu.async_copy(x_hbm.at[pl.ds(base, RC)], xb0, semx0)
    pltpu.async_copy(x_hbm.at[pl.ds(base + RC, RC)], xb1, semx1)
    cc0.wait()
    cc1.wait()

    LSH = CPAD - 128

    def _mbody(i, carry):
        c0v[pl.ds(LSH + i * 16, 16)] = (c0v[pl.ds(LSH + i * 16, 16)]
                                        + c1v[pl.ds(LSH + i * 16, 16)])
        return carry

    lax.fori_loop(0, RPT // 16, _mbody, 0)

    zero16 = jnp.zeros((16,), jnp.float32)
    for d in range(D // 16):
        accv[pl.ds(d * 16, 16)] = zero16

    def _chunk(buf, a):
        coff = LSH + a * RC

        def _gbody(g, carry):
            c16 = c0v[pl.ds(coff + g * 16, 16)]
            row0 = g * 16
            splats = [_splat(c16, r) for r in range(16)]
            for d in range(D // 16):
                terms = [splats[r] * buf[row0 + r, pl.ds(d * 16, 16)]
                         for r in range(16)]
                while len(terms) > 1:
                    terms = [terms[i] + terms[i + 1]
                             for i in range(0, len(terms), 2)]
                dd = pl.ds(d * 16, 16)
                accv[dd] = accv[dd] + terms[0]
            return carry

        lax.fori_loop(0, GR, _gbody, 0)

    def _pbody(t, carry):
        a0 = 2 * t
        pltpu.make_async_copy(x_hbm.at[pl.ds(0, RC)], xb0, semx0).wait()
        _chunk(xb0, a0)

        @pl.when(a0 + 2 < NCH)
        def _():
            pltpu.async_copy(
                x_hbm.at[pl.ds(base + (a0 + 2) * RC, RC)], xb0, semx0)

        pltpu.make_async_copy(x_hbm.at[pl.ds(0, RC)], xb1, semx1).wait()
        _chunk(xb1, a0 + 1)

        @pl.when(a0 + 3 < NCH)
        def _():
            pltpu.async_copy(
                x_hbm.at[pl.ds(base + (a0 + 3) * RC, RC)], xb1, semx1)

        return carry

    lax.fori_loop(0, NCH // 2, _pbody, 0)
    pltpu.sync_copy(accv, out_hbm.at[w])


def _mv_body(c_ref, x_ref, o_ref):
    k = pl.program_id(0)

    @pl.when(k == 0)
    def _():
        o_ref[...] = jnp.zeros_like(o_ref)

    c = c_ref[0, pl.ds(k, 1), :] + c_ref[1, pl.ds(k, 1), :]
    o_ref[...] += jnp.dot(c, x_ref[...], preferred_element_type=jnp.float32)

    @pl.when(k == NKB - 1)
    def _():
        o_ref[...] = o_ref[...] * SCALE


_matvec = pl.pallas_call(
    _mv_body,
    grid=(NKB,),
    in_specs=[
        pl.BlockSpec((NC, NKB, BK), lambda k: (0, 0, 0)),
        pl.BlockSpec((BK, D), lambda k: (k, 0)),
    ],
    out_specs=pl.BlockSpec((1, D), lambda k: (0, 0)),
    out_shape=jax.ShapeDtypeStruct((1, D), jnp.float32),
)


def kernel(x_atom_fea, node_atom_idx):
    idx = node_atom_idx.astype(jnp.int32).reshape(NW, NCHUNK, CHUNK)
    counts, cnta, cntb = _histogram(idx)
    counts3 = counts[:, :TROWS].reshape(NC, NKB, BK)
    tc = _matvec(counts3, x_atom_fea)
    sc = _scmv(cnta, cntb, x_atom_fea)
    return tc + jnp.sum(sc, axis=0, keepdims=True) * SCALE


# speedup vs baseline: 1.7728x; 1.0407x over previous
import functools

import jax
import jax.numpy as jnp
from jax import lax
from jax.experimental import pallas as pl
from jax.experimental.pallas import tpu as pltpu
from jax.experimental.pallas import tpu_sc as plsc

NC = 2
NS = 16
NW = NC * NS

N_ATOMS = 100000
D = 512
N0 = 4096
A = 24
TOTAL = N0 * A
PER_TILE = TOTAL // NW
CHUNK = 128
NCHUNK = PER_TILE // CHUNK
ZCH = 6256
TBL = NS * ZCH

SCALE = 1.0 / float(TOTAL)

TROWS = 71328
SROWS = N_ATOMS - TROWS
RPT = SROWS // NW
RC = 64
NCH = RPT // RC
GR = RC // 16
SLAB0 = (TROWS // 128) * 128
CPAD = TROWS - SLAB0
SLABSZ = TBL - SLAB0
CREAD = RPT + 128

_sc_mesh = plsc.VectorSubcoreMesh(core_axis_name="c", subcore_axis_name="s")


@functools.partial(
    pl.kernel,
    out_type=[
        jax.ShapeDtypeStruct((NC, TBL), jnp.float32),
        jax.ShapeDtypeStruct((SLABSZ,), jnp.float32),
        jax.ShapeDtypeStruct((SLABSZ,), jnp.float32),
    ],
    mesh=_sc_mesh,
    scratch_types=[
        pltpu.VMEM((NCHUNK, CHUNK), jnp.int32),
        pltpu.VMEM((CHUNK,), jnp.float32),
        pltpu.VMEM((ZCH,), jnp.float32),
        pltpu.VMEM_SHARED((TBL,), jnp.float32),
        pltpu.SemaphoreType.DMA,
        pltpu.SemaphoreType.DMA,
    ],
)
def _histogram(idx_hbm, out_hbm, outa_hbm, outb_hbm, idx_v, ones_v, zeros_v,
               table_sh, sem_idx, sem_sc):
    cid = lax.axis_index("c")
    sid = lax.axis_index("s")
    w = cid * NS + sid

    idx_cp = pltpu.async_copy(idx_hbm.at[w], idx_v, sem_idx)

    ones16 = jnp.full((16,), 1.0, jnp.float32)
    for i in range(CHUNK // 16):
        ones_v[pl.ds(i * 16, 16)] = ones16

    zero16 = jnp.zeros((16,), jnp.float32)

    def _zbody(i, carry):
        for u in range(17):
            zeros_v[pl.ds((i * 17 + u) * 16, 16)] = zero16
        return carry

    lax.fori_loop(0, ZCH // (16 * 17), _zbody, 0)

    pltpu.sync_copy(zeros_v, table_sh.at[pl.ds(sid * ZCH, ZCH)])
    idx_cp.wait()
    plsc.subcore_barrier()

    cps = [
        pltpu.async_copy(ones_v, table_sh.at[idx_v.at[j]], sem_sc, add=True)
        for j in range(NCHUNK)
    ]
    for cp in cps:
        cp.wait()
    plsc.subcore_barrier()

    @pl.when(sid == 0)
    def _():
        pltpu.sync_copy(table_sh, out_hbm.at[cid])

    @pl.when((sid == 1) & (cid == 0))
    def _():
        pltpu.sync_copy(table_sh.at[pl.ds(SLAB0, SLABSZ)], outa_hbm)

    @pl.when((sid == 1) & (cid == 1))
    def _():
        pltpu.sync_copy(table_sh.at[pl.ds(SLAB0, SLABSZ)], outb_hbm)


BK = 5944
NKB = TROWS // BK


def _splat(vec16, r):
    idx = jnp.full((16, 1), r, dtype=jnp.int32)
    return lax.gather(
        vec16, idx,
        lax.GatherDimensionNumbers(offset_dims=(), collapsed_slice_dims=(0,),
                                   start_index_map=(0,)),
        (1,), mode=lax.GatherScatterMode.PROMISE_IN_BOUNDS)


@functools.partial(
    pl.kernel,
    out_type=jax.ShapeDtypeStruct((NW, D), jnp.float32),
    mesh=_sc_mesh,
    scratch_types=[
        pltpu.VMEM((RC, D), jnp.float32),
        pltpu.VMEM((RC, D), jnp.float32),
        pltpu.VMEM((CREAD,), jnp.float32),
        pltpu.VMEM((CREAD,), jnp.float32),
        pltpu.VMEM((D,), jnp.float32),
        pltpu.SemaphoreType.DMA,
        pltpu.SemaphoreType.DMA,
        pltpu.SemaphoreType.DMA,
    ],
)
def _scmv(cnta_hbm, cntb_hbm, x_hbm, out_hbm, xb0, xb1, c0v, c1v, accv,
          semc, semx0, semx1):
    cid = lax.axis_index("c")
    sid = lax.axis_index("s")
    w = cid * NS + sid
    coff0 = w * RPT
    base = TROWS + w * RPT

    cc0 = pltpu.async_copy(cnta_hbm.at[pl.ds(coff0, CREAD)], c0v, semc)
    cc1 = pltpu.async_copy(cntb_hbm.at[pl.ds(coff0, CREAD)], c1v, semc)
    pltpu.async_copy(x_hbm.at[pl.ds(base, RC)], xb0, semx0)
    pltpu.async_copy(x_hbm.at[pl.ds(base + RC, RC)], xb1, semx1)
    cc0.wait()
    cc1.wait()

    LSH = CPAD

    def _mbody(i, carry):
        c0v[pl.ds(LSH + i * 16, 16)] = (c0v[pl.ds(LSH + i * 16, 16)]
                                        + c1v[pl.ds(LSH + i * 16, 16)])
        return carry

    lax.fori_loop(0, RPT // 16, _mbody, 0)

    zero16 = jnp.zeros((16,), jnp.float32)
    for d in range(D // 16):
        accv[pl.ds(d * 16, 16)] = zero16

    def _chunk(buf, a):
        coff = LSH + a * RC

        def _gbody(g, carry):
            c16 = c0v[pl.ds(coff + g * 16, 16)]
            row0 = g * 16
            splats = [_splat(c16, r) for r in range(16)]
            for d in range(D // 16):
                terms = [splats[r] * buf[row0 + r, pl.ds(d * 16, 16)]
                         for r in range(16)]
                while len(terms) > 1:
                    terms = [terms[i] + terms[i + 1]
                             for i in range(0, len(terms), 2)]
                dd = pl.ds(d * 16, 16)
                accv[dd] = accv[dd] + terms[0]
            return carry

        lax.fori_loop(0, GR, _gbody, 0)

    def _pbody(t, carry):
        a0 = 2 * t
        pltpu.make_async_copy(x_hbm.at[pl.ds(0, RC)], xb0, semx0).wait()
        _chunk(xb0, a0)

        @pl.when(a0 + 2 < NCH)
        def _():
            pltpu.async_copy(
                x_hbm.at[pl.ds(base + (a0 + 2) * RC, RC)], xb0, semx0)

        pltpu.make_async_copy(x_hbm.at[pl.ds(0, RC)], xb1, semx1).wait()
        _chunk(xb1, a0 + 1)

        @pl.when(a0 + 3 < NCH)
        def _():
            pltpu.async_copy(
                x_hbm.at[pl.ds(base + (a0 + 3) * RC, RC)], xb1, semx1)

        return carry

    lax.fori_loop(0, NCH // 2, _pbody, 0)
    pltpu.sync_copy(accv, out_hbm.at[w])


def _mv_body(c_ref, x_ref, o_ref):
    k = pl.program_id(0)

    @pl.when(k == 0)
    def _():
        o_ref[...] = jnp.zeros_like(o_ref)

    c = c_ref[0, pl.ds(k, 1), :] + c_ref[1, pl.ds(k, 1), :]
    o_ref[...] += jnp.dot(c, x_ref[...], preferred_element_type=jnp.float32)

    @pl.when(k == NKB - 1)
    def _():
        o_ref[...] = o_ref[...] * SCALE


_matvec = pl.pallas_call(
    _mv_body,
    grid=(NKB,),
    in_specs=[
        pl.BlockSpec((NC, NKB, BK), lambda k: (0, 0, 0)),
        pl.BlockSpec((BK, D), lambda k: (k, 0)),
    ],
    out_specs=pl.BlockSpec((1, D), lambda k: (0, 0)),
    out_shape=jax.ShapeDtypeStruct((1, D), jnp.float32),
)


def kernel(x_atom_fea, node_atom_idx):
    idx = node_atom_idx.astype(jnp.int32).reshape(NW, NCHUNK, CHUNK)
    counts, cnta, cntb = _histogram(idx)
    counts3 = counts[:, :TROWS].reshape(NC, NKB, BK)
    tc = _matvec(counts3, x_atom_fea)
    sc = _scmv(cnta, cntb, x_atom_fea)
    return tc + jnp.sum(sc, axis=0, keepdims=True) * SCALE


# speedup vs baseline: 1.8644x; 1.0516x over previous
import functools

import jax
import jax.numpy as jnp
from jax import lax
from jax.experimental import pallas as pl
from jax.experimental.pallas import tpu as pltpu
from jax.experimental.pallas import tpu_sc as plsc

NC = 2
NS = 16
NW = NC * NS

N_ATOMS = 100000
D = 512
N0 = 4096
A = 24
TOTAL = N0 * A
PER_TILE = TOTAL // NW
CHUNK = 128
NCHUNK = PER_TILE // CHUNK
ZCH = 6256
TBL = NS * ZCH

SCALE = 1.0 / float(TOTAL)

_sc_mesh = plsc.VectorSubcoreMesh(core_axis_name="c", subcore_axis_name="s")


@functools.partial(
    pl.kernel,
    out_type=jax.ShapeDtypeStruct((NC, TBL), jnp.float32),
    mesh=_sc_mesh,
    scratch_types=[
        pltpu.VMEM((NCHUNK, CHUNK), jnp.int32),
        pltpu.VMEM((CHUNK,), jnp.float32),
        pltpu.VMEM((ZCH,), jnp.float32),
        pltpu.VMEM_SHARED((TBL,), jnp.float32),
        pltpu.SemaphoreType.DMA,
        pltpu.SemaphoreType.DMA,
    ],
)
def _histogram(idx_hbm, out_hbm, idx_v, ones_v, zeros_v, table_sh,
               sem_idx, sem_sc):
    cid = lax.axis_index("c")
    sid = lax.axis_index("s")
    w = cid * NS + sid

    idx_cp = pltpu.async_copy(idx_hbm.at[w], idx_v, sem_idx)

    ones16 = jnp.full((16,), 1.0, jnp.float32)
    for i in range(CHUNK // 16):
        ones_v[pl.ds(i * 16, 16)] = ones16

    zero16 = jnp.zeros((16,), jnp.float32)

    def _zbody(i, carry):
        for u in range(17):
            zeros_v[pl.ds((i * 17 + u) * 16, 16)] = zero16
        return carry

    lax.fori_loop(0, ZCH // (16 * 17), _zbody, 0)

    pltpu.sync_copy(zeros_v, table_sh.at[pl.ds(sid * ZCH, ZCH)])
    idx_cp.wait()
    plsc.subcore_barrier()

    cps = [
        pltpu.async_copy(ones_v, table_sh.at[idx_v.at[j]], sem_sc, add=True)
        for j in range(NCHUNK)
    ]
    for cp in cps:
        cp.wait()
    plsc.subcore_barrier()

    @pl.when(sid == 0)
    def _():
        pltpu.sync_copy(table_sh, out_hbm.at[cid])


BK = 4000
NKB = N_ATOMS // BK


def _mv_body(c_ref, x_ref, o_ref):
    k = pl.program_id(0)

    @pl.when(k == 0)
    def _():
        o_ref[...] = jnp.zeros_like(o_ref)

    c = c_ref[0, pl.ds(k, 1), :] + c_ref[1, pl.ds(k, 1), :]
    o_ref[...] += jnp.dot(c, x_ref[...], preferred_element_type=jnp.float32)

    @pl.when(k == NKB - 1)
    def _():
        o_ref[...] = o_ref[...] * SCALE


_matvec = pl.pallas_call(
    _mv_body,
    grid=(NKB,),
    in_specs=[
        pl.BlockSpec((NC, NKB, BK), lambda k: (0, 0, 0)),
        pl.BlockSpec((BK, D), lambda k: (k, 0)),
    ],
    out_specs=pl.BlockSpec((1, D), lambda k: (0, 0)),
    out_shape=jax.ShapeDtypeStruct((1, D), jnp.float32),
)


def kernel(x_atom_fea, node_atom_idx):
    idx = node_atom_idx.astype(jnp.int32).reshape(NW, NCHUNK, CHUNK)
    counts = _histogram(idx)
    counts3 = counts[:, :N_ATOMS].reshape(NC, NKB, BK)
    return _matvec(counts3, x_atom_fea)
